# baseline (speedup 1.0000x reference)
import functools

import jax
import jax.numpy as jnp
from jax import lax
from jax.experimental import pallas as pl
from jax.experimental.pallas import tpu as pltpu
from jax.experimental.pallas import tpu_sc as plsc

N = 10000
E = 320000
D = 128
H = D // 2

NC = 2
NS = 16
CHUNK = 32
EPT = E // NS
NPAD = 10240
RPT = NPAD // NS
ZROWS = 128
MAGIC = 0x7EF311C7


def _a1_body(h_ref, aw_ref, ab_ref, bw_ref, bb_ref, dw_ref, db_ref,
             ew_ref, eb_ref, ah_ref, db_tab_ref, eh_tab_ref):
    hb = h_ref[...]
    dn = (((1,), (1,)), ((), ()))
    ah = lax.dot_general(hb, aw_ref[...], dn,
                         preferred_element_type=jnp.float32) + ab_ref[...]
    bh = lax.dot_general(hb, bw_ref[...], dn,
                         preferred_element_type=jnp.float32) + bb_ref[...]
    dh = lax.dot_general(hb, dw_ref[...], dn,
                         preferred_element_type=jnp.float32) + db_ref[...]
    eh = lax.dot_general(hb, ew_ref[...], dn,
                         preferred_element_type=jnp.float32) + eb_ref[...]
    ah_ref[...] = ah
    z = jnp.zeros_like(eh[:, :H])
    db_tab_ref[0] = jnp.concatenate([dh[:, :H], bh[:, :H]], axis=1)
    db_tab_ref[1] = jnp.concatenate([dh[:, H:], bh[:, H:]], axis=1)
    eh_tab_ref[0] = jnp.concatenate([eh[:, :H], z], axis=1)
    eh_tab_ref[1] = jnp.concatenate([eh[:, H:], z], axis=1)


def _node_transforms(h, Aw, Ab, Bw, Bb, Dw, Db, Ew, Eb):
    R = 1000
    grid = (N // R,)
    wspec = pl.BlockSpec((D, D), lambda i: (0, 0))
    bspec = pl.BlockSpec((D,), lambda i: (0,))
    return pl.pallas_call(
        _a1_body,
        grid=grid,
        in_specs=[pl.BlockSpec((R, D), lambda i: (i, 0)),
                  wspec, bspec, wspec, bspec, wspec, bspec, wspec, bspec],
        out_specs=[pl.BlockSpec((R, D), lambda i: (i, 0)),
                   pl.BlockSpec((2, R, D), lambda i: (0, i, 0)),
                   pl.BlockSpec((2, R, D), lambda i: (0, i, 0))],
        out_shape=[jax.ShapeDtypeStruct((N, D), jnp.float32),
                   jax.ShapeDtypeStruct((2, N, D), jnp.float32),
                   jax.ShapeDtypeStruct((2, N, D), jnp.float32)],
    )(h, Aw, Ab, Bw, Bb, Dw, Db, Ew, Eb)


def _a2_body(e2_ref, wp_ref, cbp_ref, ce_ref):
    e2 = e2_ref[...]
    ce_ref[0] = lax.dot_general(e2, wp_ref[0], (((1,), (0,)), ((), ())),
                                preferred_element_type=jnp.float32) + cbp_ref[0]
    ce_ref[1] = lax.dot_general(e2, wp_ref[1], (((1,), (0,)), ((), ())),
                                preferred_element_type=jnp.float32) + cbp_ref[1]


def _edge_transform(e2, Wp, Cbp):
    T2 = 2000
    return pl.pallas_call(
        _a2_body,
        grid=(E // 2 // T2,),
        in_specs=[pl.BlockSpec((T2, 2 * D), lambda i: (i, 0)),
                  pl.BlockSpec((2, 2 * D, D), lambda i: (0, 0, 0)),
                  pl.BlockSpec((2, D), lambda i: (0, 0))],
        out_specs=pl.BlockSpec((2, T2, D), lambda i: (0, i, 0)),
        out_shape=jax.ShapeDtypeStruct((2, E // 2, D), jnp.float32),
    )(e2, Wp, Cbp)


def _sc_body(db_hbm, eh_hbm, ce_hbm, src_hbm, dst_hbm,
             enew_hbm, acc_hbm, ssum_hbm, ssq_hbm,
             acc,
             src_v0, dst_v0, dstg_v0, db_b0, eh_b0, ce_b0, out_b0, ms_b0,
             srcB0, dstB0, sem_i0,
             src_v1, dst_v1, dstg_v1, db_b1, eh_b1, ce_b1, out_b1, ms_b1,
             srcB1, dstB1, sem_i1,
             sum_st, sq_st, sem_g0, sem_g1, sem_w0, sem_w1):
    c = lax.axis_index("c")
    s = lax.axis_index("s")
    cN = c * N
    cP = c * NPAD
    cE2 = c * (E // 2)

    bufs = ((src_v0, dst_v0, dstg_v0, db_b0, eh_b0, ce_b0, out_b0, ms_b0,
             sem_g0, sem_w0),
            (src_v1, dst_v1, dstg_v1, db_b1, eh_b1, ce_b1, out_b1, ms_b1,
             sem_g1, sem_w1))
    ibufs = ((srcB0, dstB0, sem_i0), (srcB1, dstB1, sem_i1))

    def zb_row(r, _):
        for j in range(D // 16):
            ms_b0[r, pl.ds(j * 16, 16)] = jnp.zeros((16,), jnp.float32)
        return 0
    lax.fori_loop(0, CHUNK, zb_row, 0)
    for k in range(RPT // CHUNK):
        pltpu.sync_copy(ms_b0, acc.at[pl.ds(s * RPT + k * CHUNK, CHUNK)])
    plsc.subcore_barrier()

    base0 = s * EPT
    base20 = s * (EPT // 2)

    def issue_idx(b, k):
        sb, db_, si = ibufs[b]
        base = base0 + k * CHUNK
        pltpu.async_copy(src_hbm.at[pl.ds(base, CHUNK)], sb, si)
        pltpu.async_copy(dst_hbm.at[pl.ds(base, CHUNK)], db_, si)

    def wait_idx(b):
        sb, db_, si = ibufs[b]
        pltpu.make_async_copy(src_hbm.at[pl.ds(0, CHUNK)], sb, si).wait()
        pltpu.make_async_copy(dst_hbm.at[pl.ds(0, CHUNK)], db_, si).wait()

    def issue_loads(b, k):
        sv, dv, gv, bdb, beh, bce, _, _, sg, _ = bufs[b]
        sb, db_, si = ibufs[b]
        base2 = base20 + k * (CHUNK // 2)
        wait_idx(b)
        for j in range(CHUNK // 16):
            sl = pl.ds(j * 16, 16)
            sv[sl] = sb[sl] + cN
            rd = db_[sl]
            dv[sl] = rd
            gv[sl] = rd + cN
        pltpu.async_copy(db_hbm.at[sv], bdb, sg)
        pltpu.async_copy(eh_hbm.at[gv], beh, sg)
        pltpu.async_copy(ce_hbm.at[pl.ds(cE2 + base2, CHUNK // 2)], bce, sg)

    def wait_loads(b, k):
        sv, dv, gv, bdb, beh, bce, _, _, sg, _ = bufs[b]
        base2 = base20 + k * (CHUNK // 2)
        pltpu.make_async_copy(db_hbm.at[sv], bdb, sg).wait()
        pltpu.make_async_copy(eh_hbm.at[gv], beh, sg).wait()
        pltpu.make_async_copy(
            ce_hbm.at[pl.ds(cE2 + base2, CHUNK // 2)], bce, sg).wait()

    def finish_chunk(b, k):
        _, dv, _, _, _, _, bout, bms, _, sw = bufs[b]
        base2 = base20 + k * (CHUNK // 2)
        pltpu.sync_copy(bms, acc.at[dv], add=True)
        pltpu.async_copy(bout, enew_hbm.at[pl.ds(cE2 + base2, CHUNK // 2)],
                         sw)

    def drain_enew(b):
        _, _, _, _, _, _, bout, _, _, sw = bufs[b]
        pltpu.make_async_copy(bout, enew_hbm.at[pl.ds(0, CHUNK // 2)],
                              sw).wait()

    def compute(b, st):
        _, _, _, bdb, beh, bce, bout, bms, _, _ = bufs[b]

        @plsc.parallel_loop(0, CHUNK // 2, 1, unroll=4, carry=tuple(st))
        def pair_body(rr, st):
            st = list(st)
            for p in range(2):
                r = 2 * rr + p
                for j in range(H // 16):
                    sl_de = pl.ds(j * 16, 16)
                    sl_b = pl.ds(H + j * 16, 16)
                    sl_ce = pl.ds(p * H + j * 16, 16)
                    x = bdb[r, sl_de] + beh[r, sl_de] + bce[rr, sl_ce]
                    bout[rr, sl_ce] = x
                    st[j] = st[j] + x
                    st[4 + j] = st[4 + j] + x * x
                    t = jnp.exp(-jnp.maximum(x, -87.0))
                    y = 1.0 + t
                    iy = lax.bitcast_convert_type(y, jnp.int32)
                    rc = lax.bitcast_convert_type(MAGIC - iy, jnp.float32)
                    rc = rc * (2.0 - y * rc)
                    sg = rc * (2.0 - y * rc)
                    bms[r, sl_b] = sg
                    bms[r, sl_de] = bdb[r, sl_b] * sg
            return tuple(st)
        return pair_body

    NCHUNK = EPT // CHUNK
    NG2 = NCHUNK // 2

    def body2(g, st):
        k0 = 2 * g
        wait_loads(0, k0)

        @pl.when(g > 0)
        def _():
            drain_enew(0)
        st = compute(0, st)
        finish_chunk(0, k0)

        @pl.when(k0 + 2 < NCHUNK)
        def _():
            issue_loads(0, k0 + 2)

        @pl.when(k0 + 4 < NCHUNK)
        def _():
            issue_idx(0, k0 + 4)
        wait_loads(1, k0 + 1)

        @pl.when(g > 0)
        def _():
            drain_enew(1)
        st = compute(1, st)
        finish_chunk(1, k0 + 1)

        @pl.when(k0 + 3 < NCHUNK)
        def _():
            issue_loads(1, k0 + 3)

        @pl.when(k0 + 5 < NCHUNK)
        def _():
            issue_idx(1, k0 + 5)
        return st

    issue_idx(0, 0)
    issue_idx(1, 1)
    issue_loads(0, 0)
    issue_idx(0, 2)
    issue_loads(1, 1)
    issue_idx(1, 3)
    zero16 = jnp.zeros((16,), jnp.float32)
    stats = lax.fori_loop(0, NG2, body2, (zero16,) * 8)
    drain_enew(0)
    if NCHUNK % 2:
        tk = 2 * NG2
        wait_loads(0, tk)
        stats = compute(0, stats)
        finish_chunk(0, tk)
        drain_enew(0)
    drain_enew(1)

    def st_row(r, _):
        for j in range(D // 16):
            sum_st[r, pl.ds(j * 16, 16)] = jnp.zeros((16,), jnp.float32)
            sq_st[r, pl.ds(j * 16, 16)] = jnp.zeros((16,), jnp.float32)
        return 0
    lax.fori_loop(0, 8, st_row, 0)
    for j in range(H // 16):
        sum_st[0, pl.ds(j * 16, 16)] = stats[j]
        sq_st[0, pl.ds(j * 16, 16)] = stats[4 + j]
    row = (c * NS + s) * 8
    pltpu.sync_copy(sum_st, ssum_hbm.at[pl.ds(row, 8)])
    pltpu.sync_copy(sq_st, ssq_hbm.at[pl.ds(row, 8)])

    plsc.subcore_barrier()
    pltpu.sync_copy(acc.at[pl.ds(s * RPT, RPT)],
                    acc_hbm.at[pl.ds(cP + s * RPT, RPT)])


def _sc_stage(db, eh, ce, src, dst):
    mesh = plsc.VectorSubcoreMesh(core_axis_name="c", subcore_axis_name="s",
                                  num_cores=NC, num_subcores=NS)
    f = pl.kernel(
        _sc_body,
        out_type=[jax.ShapeDtypeStruct((E, D), jnp.float32),
                  jax.ShapeDtypeStruct((2 * NPAD, D), jnp.float32),
                  jax.ShapeDtypeStruct((NC * NS * 8, D), jnp.float32),
                  jax.ShapeDtypeStruct((NC * NS * 8, D), jnp.float32)],
        mesh=mesh,
        scratch_types=(
            [pltpu.VMEM_SHARED((NPAD, D), jnp.float32)]
            + 2 * [pltpu.VMEM((CHUNK,), jnp.int32),
                   pltpu.VMEM((CHUNK,), jnp.int32),
                   pltpu.VMEM((CHUNK,), jnp.int32),
                   pltpu.VMEM((CHUNK, D), jnp.float32),
                   pltpu.VMEM((CHUNK, D), jnp.float32),
                   pltpu.VMEM((CHUNK // 2, D), jnp.float32),
                   pltpu.VMEM((CHUNK // 2, D), jnp.float32),
                   pltpu.VMEM((CHUNK, D), jnp.float32),
                   pltpu.VMEM((CHUNK,), jnp.int32),
                   pltpu.VMEM((CHUNK,), jnp.int32),
                   pltpu.SemaphoreType.DMA]
            + [pltpu.VMEM((8, D), jnp.float32),
               pltpu.VMEM((8, D), jnp.float32),
               pltpu.SemaphoreType.DMA,
               pltpu.SemaphoreType.DMA,
               pltpu.SemaphoreType.DMA,
               pltpu.SemaphoreType.DMA]
        ),
    )
    return f(db, eh, ce, src, dst)


def _c1_body(h_ref, ah_ref, acc_ref, sc_ref, bi_ref, out_ref):
    sh = jnp.concatenate([acc_ref[0, :N, :H], acc_ref[1, :N, :H]], axis=1)
    ss = jnp.concatenate([acc_ref[0, :N, H:], acc_ref[1, :N, H:]], axis=1)
    hn = ah_ref[...] + sh / (ss + 1e-6)
    mu = jnp.mean(hn, axis=0)
    var = jnp.mean(hn * hn, axis=0) - mu * mu
    bn = (hn - mu) / jnp.sqrt(var + 1e-5) * sc_ref[...] + bi_ref[...]
    out_ref[...] = h_ref[...] + jnp.maximum(bn, 0.0)


def _finalize_h(h, ah, acc, scale, bias):
    return pl.pallas_call(
        _c1_body,
        out_shape=jax.ShapeDtypeStruct((N, D), jnp.float32),
    )(h, ah, acc, scale, bias)


def _c2_body(e2_ref, en_ref, ssum_ref, ssq_ref, sc_ref, bi_ref, out_ref):
    en0 = en_ref[0]
    en1 = en_ref[1]
    en = jnp.concatenate([en0[:, :H], en1[:, :H], en0[:, H:], en1[:, H:]],
                         axis=1)
    hrows = NS * 8
    s0 = jnp.sum(ssum_ref[:hrows, :H], axis=0)
    s1 = jnp.sum(ssum_ref[hrows:, :H], axis=0)
    q0 = jnp.sum(ssq_ref[:hrows, :H], axis=0)
    q1 = jnp.sum(ssq_ref[hrows:, :H], axis=0)
    mu = jnp.concatenate([s0, s1]) * (1.0 / E)
    msq = jnp.concatenate([q0, q1]) * (1.0 / E)
    var = msq - mu * mu
    rstd = 1.0 / jnp.sqrt(var + 1e-5)
    mu2 = jnp.concatenate([mu, mu])
    rstd2 = jnp.concatenate([rstd, rstd])
    sc2 = jnp.concatenate([sc_ref[...], sc_ref[...]])
    bi2 = jnp.concatenate([bi_ref[...], bi_ref[...]])
    bn = (en - mu2) * rstd2 * sc2 + bi2
    out_ref[...] = e2_ref[...] + jnp.maximum(bn, 0.0)


def _finalize_e(e2, enew, ssum, ssq, scale, bias):
    T2 = 2000
    return pl.pallas_call(
        _c2_body,
        grid=(E // 2 // T2,),
        in_specs=[pl.BlockSpec((T2, 2 * D), lambda i: (i, 0)),
                  pl.BlockSpec((2, T2, D), lambda i: (0, i, 0)),
                  pl.BlockSpec((NC * NS * 8, D), lambda i: (0, 0)),
                  pl.BlockSpec((NC * NS * 8, D), lambda i: (0, 0)),
                  pl.BlockSpec((D,), lambda i: (0,)),
                  pl.BlockSpec((D,), lambda i: (0,))],
        out_specs=pl.BlockSpec((T2, 2 * D), lambda i: (i, 0)),
        out_shape=jax.ShapeDtypeStruct((E // 2, 2 * D), jnp.float32),
    )(e2, enew, ssum, ssq, scale, bias)


@jax.jit
def _run(h, e, edge_index, Aw, Ab, Bw, Bb, Cw, Cb, Dw, Db, Ew, Eb,
         bnh_scale, bnh_bias, bne_scale, bne_bias):
    src = edge_index[0].astype(jnp.int32)
    dst = edge_index[1].astype(jnp.int32)

    wt = Cw.T
    z = jnp.zeros((D, H), jnp.float32)
    wp_list = []
    cbp_list = []
    for c in range(2):
        blk = wt[:, c * H:(c + 1) * H]
        top = jnp.concatenate([blk, z], axis=1)
        bot = jnp.concatenate([z, blk], axis=1)
        wp_list.append(jnp.concatenate([top, bot], axis=0))
        cbp_list.append(jnp.tile(Cb[c * H:(c + 1) * H], 2))
    Wp = jnp.stack(wp_list)
    Cbp = jnp.stack(cbp_list)
    e2 = e.reshape(E // 2, 2 * D)

    ah, db_tab, eh_tab = _node_transforms(h, Aw, Ab, Bw, Bb, Dw, Db, Ew, Eb)
    ce = _edge_transform(e2, Wp, Cbp)

    enew, accs, ssum, ssq = _sc_stage(db_tab.reshape(2 * N, D),
                                      eh_tab.reshape(2 * N, D),
                                      ce.reshape(E, D), src, dst)

    h_out = _finalize_h(h, ah, accs.reshape(2, NPAD, D), bnh_scale, bnh_bias)
    e_out2 = _finalize_e(e2, enew.reshape(2, E // 2, D), ssum, ssq,
                         bne_scale, bne_bias)
    return h_out, e_out2.reshape(E, D)


def kernel(h, e, edge_index, Aw, Ab, Bw, Bb, Cw, Cb, Dw, Db, Ew, Eb,
           bnh_scale, bnh_bias, bne_scale, bne_bias):
    return _run(h, e, edge_index, Aw, Ab, Bw, Bb, Cw, Cb, Dw, Db, Ew, Eb,
                bnh_scale, bnh_bias, bne_scale, bne_bias)


# speedup vs baseline: 1.9666x; 1.9666x over previous
import functools

import jax
import jax.numpy as jnp
from jax import lax
from jax.experimental import pallas as pl
from jax.experimental.pallas import tpu as pltpu
from jax.experimental.pallas import tpu_sc as plsc

N = 10000
E = 320000
D = 128
H = D // 2

NC = 2
NS = 16
CHUNK = 32
EPT = E // NS
NPAD = 10240
RPT = NPAD // NS
ZROWS = 128
MAGIC = 0x7EF311C7


def _a1_body(h_ref, aw_ref, ab_ref, bw_ref, bb_ref, dw_ref, db_ref,
             ew_ref, eb_ref, ah_ref, db_tab_ref, eh_tab_ref):
    hb = h_ref[...]
    dn = (((1,), (1,)), ((), ()))
    ah = lax.dot_general(hb, aw_ref[...], dn,
                         preferred_element_type=jnp.float32) + ab_ref[...]
    bh = lax.dot_general(hb, bw_ref[...], dn,
                         preferred_element_type=jnp.float32) + bb_ref[...]
    dh = lax.dot_general(hb, dw_ref[...], dn,
                         preferred_element_type=jnp.float32) + db_ref[...]
    eh = lax.dot_general(hb, ew_ref[...], dn,
                         preferred_element_type=jnp.float32) + eb_ref[...]
    ah_ref[...] = ah
    z = jnp.zeros_like(eh[:, :H])
    db_tab_ref[0] = jnp.concatenate([dh[:, :H], bh[:, :H]], axis=1)
    db_tab_ref[1] = jnp.concatenate([dh[:, H:], bh[:, H:]], axis=1)
    eh_tab_ref[0] = jnp.concatenate([eh[:, :H], z], axis=1)
    eh_tab_ref[1] = jnp.concatenate([eh[:, H:], z], axis=1)


def _node_transforms(h, Aw, Ab, Bw, Bb, Dw, Db, Ew, Eb):
    R = 1000
    grid = (N // R,)
    wspec = pl.BlockSpec((D, D), lambda i: (0, 0))
    bspec = pl.BlockSpec((D,), lambda i: (0,))
    return pl.pallas_call(
        _a1_body,
        grid=grid,
        in_specs=[pl.BlockSpec((R, D), lambda i: (i, 0)),
                  wspec, bspec, wspec, bspec, wspec, bspec, wspec, bspec],
        out_specs=[pl.BlockSpec((R, D), lambda i: (i, 0)),
                   pl.BlockSpec((2, R, D), lambda i: (0, i, 0)),
                   pl.BlockSpec((2, R, D), lambda i: (0, i, 0))],
        out_shape=[jax.ShapeDtypeStruct((N, D), jnp.float32),
                   jax.ShapeDtypeStruct((2, N, D), jnp.float32),
                   jax.ShapeDtypeStruct((2, N, D), jnp.float32)],
    )(h, Aw, Ab, Bw, Bb, Dw, Db, Ew, Eb)


def _a2_body(e2_ref, wp_ref, cbp_ref, ce_ref):
    e2 = e2_ref[...]
    ce_ref[0] = lax.dot_general(e2, wp_ref[0], (((1,), (0,)), ((), ())),
                                preferred_element_type=jnp.float32) + cbp_ref[0]
    ce_ref[1] = lax.dot_general(e2, wp_ref[1], (((1,), (0,)), ((), ())),
                                preferred_element_type=jnp.float32) + cbp_ref[1]


def _edge_transform(e2, Wp, Cbp):
    T2 = 2000
    return pl.pallas_call(
        _a2_body,
        grid=(E // 2 // T2,),
        in_specs=[pl.BlockSpec((T2, 2 * D), lambda i: (i, 0)),
                  pl.BlockSpec((2, 2 * D, D), lambda i: (0, 0, 0)),
                  pl.BlockSpec((2, D), lambda i: (0, 0))],
        out_specs=pl.BlockSpec((2, T2, D), lambda i: (0, i, 0)),
        out_shape=jax.ShapeDtypeStruct((2, E // 2, D), jnp.float32),
    )(e2, Wp, Cbp)


def _sc_body(db_hbm, eh_hbm, ce_hbm, src_hbm, dst_hbm,
             enew_hbm, acc_hbm,
             acc,
             src_v0, dst_v0, dstg_v0, db_b0, eh_b0, ce_b0, out_b0, ms_b0,
             srcB0, dstB0, sem_i0,
             src_v1, dst_v1, dstg_v1, db_b1, eh_b1, ce_b1, out_b1, ms_b1,
             srcB1, dstB1, sem_i1,
             sem_g0, sem_g1, sem_w0, sem_w1):
    c = lax.axis_index("c")
    s = lax.axis_index("s")
    cN = c * N
    cP = c * NPAD
    cE2 = c * (E // 2)

    bufs = ((src_v0, dst_v0, dstg_v0, db_b0, eh_b0, ce_b0, out_b0, ms_b0,
             sem_g0, sem_w0),
            (src_v1, dst_v1, dstg_v1, db_b1, eh_b1, ce_b1, out_b1, ms_b1,
             sem_g1, sem_w1))
    ibufs = ((srcB0, dstB0, sem_i0), (srcB1, dstB1, sem_i1))

    def zb_row(r, _):
        for j in range(D // 16):
            ms_b0[r, pl.ds(j * 16, 16)] = jnp.zeros((16,), jnp.float32)
        return 0
    lax.fori_loop(0, CHUNK, zb_row, 0)
    for k in range(RPT // CHUNK):
        pltpu.sync_copy(ms_b0, acc.at[pl.ds(s * RPT + k * CHUNK, CHUNK)])
    plsc.subcore_barrier()

    base0 = s * EPT
    base20 = s * (EPT // 2)

    def issue_idx(b, k):
        sb, db_, si = ibufs[b]
        base = base0 + k * CHUNK
        pltpu.async_copy(src_hbm.at[pl.ds(base, CHUNK)], sb, si)
        pltpu.async_copy(dst_hbm.at[pl.ds(base, CHUNK)], db_, si)

    def wait_idx(b):
        sb, db_, si = ibufs[b]
        pltpu.make_async_copy(src_hbm.at[pl.ds(0, CHUNK)], sb, si).wait()
        pltpu.make_async_copy(dst_hbm.at[pl.ds(0, CHUNK)], db_, si).wait()

    def issue_loads(b, k):
        sv, dv, gv, bdb, beh, bce, _, _, sg, _ = bufs[b]
        sb, db_, si = ibufs[b]
        base2 = base20 + k * (CHUNK // 2)
        wait_idx(b)
        for j in range(CHUNK // 16):
            sl = pl.ds(j * 16, 16)
            sv[sl] = sb[sl] + cN
            rd = db_[sl]
            dv[sl] = rd
            gv[sl] = rd + cN
        pltpu.async_copy(db_hbm.at[sv], bdb, sg)
        pltpu.async_copy(eh_hbm.at[gv], beh, sg)
        pltpu.async_copy(ce_hbm.at[pl.ds(cE2 + base2, CHUNK // 2)], bce, sg)

    def wait_loads(b, k):
        sv, dv, gv, bdb, beh, bce, _, _, sg, _ = bufs[b]
        base2 = base20 + k * (CHUNK // 2)
        pltpu.make_async_copy(db_hbm.at[sv], bdb, sg).wait()
        pltpu.make_async_copy(eh_hbm.at[gv], beh, sg).wait()
        pltpu.make_async_copy(
            ce_hbm.at[pl.ds(cE2 + base2, CHUNK // 2)], bce, sg).wait()

    def finish_chunk(b, k):
        _, dv, _, _, _, _, bout, bms, _, sw = bufs[b]
        base2 = base20 + k * (CHUNK // 2)
        pltpu.sync_copy(bms, acc.at[dv], add=True)
        pltpu.async_copy(bout, enew_hbm.at[pl.ds(cE2 + base2, CHUNK // 2)],
                         sw)

    def drain_enew(b):
        _, _, _, _, _, _, bout, _, _, sw = bufs[b]
        pltpu.make_async_copy(bout, enew_hbm.at[pl.ds(0, CHUNK // 2)],
                              sw).wait()

    def compute(b):
        _, _, _, bdb, beh, bce, bout, bms, _, _ = bufs[b]

        @plsc.parallel_loop(0, CHUNK // 2, 1, unroll=4)
        def pair_body(rr):
            for p in range(2):
                r = 2 * rr + p
                for j in range(H // 16):
                    sl_de = pl.ds(j * 16, 16)
                    sl_b = pl.ds(H + j * 16, 16)
                    sl_ce = pl.ds(p * H + j * 16, 16)
                    x = bdb[r, sl_de] + beh[r, sl_de] + bce[rr, sl_ce]
                    bout[rr, sl_ce] = x
                    sg = 1.0 / (1.0 + jnp.exp(-x))
                    bms[r, sl_b] = sg
                    bms[r, sl_de] = bdb[r, sl_b] * sg

    NCHUNK = EPT // CHUNK
    NG2 = NCHUNK // 2

    def body2(g, _):
        k0 = 2 * g
        wait_loads(0, k0)

        @pl.when(g > 0)
        def _():
            drain_enew(0)
        compute(0)
        finish_chunk(0, k0)

        @pl.when(k0 + 2 < NCHUNK)
        def _():
            issue_loads(0, k0 + 2)

        @pl.when(k0 + 4 < NCHUNK)
        def _():
            issue_idx(0, k0 + 4)
        wait_loads(1, k0 + 1)

        @pl.when(g > 0)
        def _():
            drain_enew(1)
        compute(1)
        finish_chunk(1, k0 + 1)

        @pl.when(k0 + 3 < NCHUNK)
        def _():
            issue_loads(1, k0 + 3)

        @pl.when(k0 + 5 < NCHUNK)
        def _():
            issue_idx(1, k0 + 5)
        return 0

    issue_idx(0, 0)
    issue_idx(1, 1)
    issue_loads(0, 0)
    issue_idx(0, 2)
    issue_loads(1, 1)
    issue_idx(1, 3)
    lax.fori_loop(0, NG2, body2, 0)
    drain_enew(0)
    if NCHUNK % 2:
        tk = 2 * NG2
        wait_loads(0, tk)
        compute(0)
        finish_chunk(0, tk)
        drain_enew(0)
    drain_enew(1)

    plsc.subcore_barrier()
    pltpu.sync_copy(acc.at[pl.ds(s * RPT, RPT)],
                    acc_hbm.at[pl.ds(cP + s * RPT, RPT)])


def _sc_stage(db, eh, ce, src, dst):
    mesh = plsc.VectorSubcoreMesh(core_axis_name="c", subcore_axis_name="s",
                                  num_cores=NC, num_subcores=NS)
    f = pl.kernel(
        _sc_body,
        out_type=[jax.ShapeDtypeStruct((E, D), jnp.float32),
                  jax.ShapeDtypeStruct((2 * NPAD, D), jnp.float32)],
        mesh=mesh,
        scratch_types=(
            [pltpu.VMEM_SHARED((NPAD, D), jnp.float32)]
            + 2 * [pltpu.VMEM((CHUNK,), jnp.int32),
                   pltpu.VMEM((CHUNK,), jnp.int32),
                   pltpu.VMEM((CHUNK,), jnp.int32),
                   pltpu.VMEM((CHUNK, D), jnp.float32),
                   pltpu.VMEM((CHUNK, D), jnp.float32),
                   pltpu.VMEM((CHUNK // 2, D), jnp.float32),
                   pltpu.VMEM((CHUNK // 2, D), jnp.float32),
                   pltpu.VMEM((CHUNK, D), jnp.float32),
                   pltpu.VMEM((CHUNK,), jnp.int32),
                   pltpu.VMEM((CHUNK,), jnp.int32),
                   pltpu.SemaphoreType.DMA]
            + [pltpu.SemaphoreType.DMA,
               pltpu.SemaphoreType.DMA,
               pltpu.SemaphoreType.DMA,
               pltpu.SemaphoreType.DMA]
        ),
    )
    return f(db, eh, ce, src, dst)


def _c1_body(h_ref, ah_ref, acc_ref, sc_ref, bi_ref, out_ref):
    sh = jnp.concatenate([acc_ref[0, :N, :H], acc_ref[1, :N, :H]], axis=1)
    ss = jnp.concatenate([acc_ref[0, :N, H:], acc_ref[1, :N, H:]], axis=1)
    hn = ah_ref[...] + sh / (ss + 1e-6)
    mu = jnp.mean(hn, axis=0)
    var = jnp.mean(hn * hn, axis=0) - mu * mu
    bn = (hn - mu) / jnp.sqrt(var + 1e-5) * sc_ref[...] + bi_ref[...]
    out_ref[...] = h_ref[...] + jnp.maximum(bn, 0.0)


def _finalize_h(h, ah, acc, scale, bias):
    return pl.pallas_call(
        _c1_body,
        out_shape=jax.ShapeDtypeStruct((N, D), jnp.float32),
    )(h, ah, acc, scale, bias)


def _c2a_body(en_ref, st_ref):
    i = pl.program_id(0)

    @pl.when(i == 0)
    def _():
        st_ref[...] = jnp.zeros_like(st_ref)
    en0 = en_ref[0]
    en1 = en_ref[1]
    st_ref[0, :D] += jnp.sum(en0, axis=0)
    st_ref[0, D:] += jnp.sum(en1, axis=0)
    st_ref[1, :D] += jnp.sum(en0 * en0, axis=0)
    st_ref[1, D:] += jnp.sum(en1 * en1, axis=0)


def _e_stats(enew):
    T2 = 4000
    return pl.pallas_call(
        _c2a_body,
        grid=(E // 2 // T2,),
        in_specs=[pl.BlockSpec((2, T2, D), lambda i: (0, i, 0))],
        out_specs=pl.BlockSpec((8, 2 * D), lambda i: (0, 0)),
        out_shape=jax.ShapeDtypeStruct((8, 2 * D), jnp.float32),
    )(enew)


def _c2_body(e2_ref, en_ref, st_ref, sc_ref, bi_ref, out_ref):
    en0 = en_ref[0]
    en1 = en_ref[1]
    en = jnp.concatenate([en0[:, :H], en1[:, :H], en0[:, H:], en1[:, H:]],
                         axis=1)
    s0 = st_ref[0, :D]
    s1 = st_ref[0, D:]
    q0 = st_ref[1, :D]
    q1 = st_ref[1, D:]
    mu = jnp.concatenate([s0[:H] + s0[H:], s1[:H] + s1[H:]]) * (1.0 / E)
    msq = jnp.concatenate([q0[:H] + q0[H:], q1[:H] + q1[H:]]) * (1.0 / E)
    var = msq - mu * mu
    rstd = 1.0 / jnp.sqrt(var + 1e-5)
    mu2 = jnp.concatenate([mu, mu])
    rstd2 = jnp.concatenate([rstd, rstd])
    sc2 = jnp.concatenate([sc_ref[...], sc_ref[...]])
    bi2 = jnp.concatenate([bi_ref[...], bi_ref[...]])
    bn = (en - mu2) * rstd2 * sc2 + bi2
    out_ref[...] = e2_ref[...] + jnp.maximum(bn, 0.0)


def _finalize_e(e2, enew, stats, scale, bias):
    T2 = 2000
    return pl.pallas_call(
        _c2_body,
        grid=(E // 2 // T2,),
        in_specs=[pl.BlockSpec((T2, 2 * D), lambda i: (i, 0)),
                  pl.BlockSpec((2, T2, D), lambda i: (0, i, 0)),
                  pl.BlockSpec((8, 2 * D), lambda i: (0, 0)),
                  pl.BlockSpec((D,), lambda i: (0,)),
                  pl.BlockSpec((D,), lambda i: (0,))],
        out_specs=pl.BlockSpec((T2, 2 * D), lambda i: (i, 0)),
        out_shape=jax.ShapeDtypeStruct((E // 2, 2 * D), jnp.float32),
    )(e2, enew, stats, scale, bias)


@jax.jit
def _run(h, e, edge_index, Aw, Ab, Bw, Bb, Cw, Cb, Dw, Db, Ew, Eb,
         bnh_scale, bnh_bias, bne_scale, bne_bias):
    src = edge_index[0].astype(jnp.int32)
    dst = edge_index[1].astype(jnp.int32)

    wt = Cw.T
    z = jnp.zeros((D, H), jnp.float32)
    wp_list = []
    cbp_list = []
    for c in range(2):
        blk = wt[:, c * H:(c + 1) * H]
        top = jnp.concatenate([blk, z], axis=1)
        bot = jnp.concatenate([z, blk], axis=1)
        wp_list.append(jnp.concatenate([top, bot], axis=0))
        cbp_list.append(jnp.tile(Cb[c * H:(c + 1) * H], 2))
    Wp = jnp.stack(wp_list)
    Cbp = jnp.stack(cbp_list)
    e2 = e.reshape(E // 2, 2 * D)

    ah, db_tab, eh_tab = _node_transforms(h, Aw, Ab, Bw, Bb, Dw, Db, Ew, Eb)
    ce = _edge_transform(e2, Wp, Cbp)

    enew, accs = _sc_stage(db_tab.reshape(2 * N, D),
                           eh_tab.reshape(2 * N, D),
                           ce.reshape(E, D), src, dst)

    enew3 = enew.reshape(2, E // 2, D)
    stats = _e_stats(enew3)
    h_out = _finalize_h(h, ah, accs.reshape(2, NPAD, D), bnh_scale, bnh_bias)
    e_out2 = _finalize_e(e2, enew3, stats, bne_scale, bne_bias)
    return h_out, e_out2.reshape(E, D)


def kernel(h, e, edge_index, Aw, Ab, Bw, Bb, Cw, Cb, Dw, Db, Ew, Eb,
           bnh_scale, bnh_bias, bne_scale, bne_bias):
    return _run(h, e, edge_index, Aw, Ab, Bw, Bb, Cw, Cb, Dw, Db, Ew, Eb,
                bnh_scale, bnh_bias, bne_scale, bne_bias)


# speedup vs baseline: 1.9698x; 1.0016x over previous
import functools

import jax
import jax.numpy as jnp
from jax import lax
from jax.experimental import pallas as pl
from jax.experimental.pallas import tpu as pltpu
from jax.experimental.pallas import tpu_sc as plsc

N = 10000
E = 320000
D = 128
H = D // 2

NC = 2
NS = 16
CHUNK = 32
EPT = E // NS
NPAD = 10240
RPT = NPAD // NS
ZROWS = 128
MAGIC = 0x7EF311C7


def _a1_body(h_ref, aw_ref, ab_ref, bw_ref, bb_ref, dw_ref, db_ref,
             ew_ref, eb_ref, ah_ref, db_tab_ref, eh_tab_ref):
    hb = h_ref[...]
    dn = (((1,), (1,)), ((), ()))
    ah = lax.dot_general(hb, aw_ref[...], dn,
                         preferred_element_type=jnp.float32) + ab_ref[...]
    bh = lax.dot_general(hb, bw_ref[...], dn,
                         preferred_element_type=jnp.float32) + bb_ref[...]
    dh = lax.dot_general(hb, dw_ref[...], dn,
                         preferred_element_type=jnp.float32) + db_ref[...]
    eh = lax.dot_general(hb, ew_ref[...], dn,
                         preferred_element_type=jnp.float32) + eb_ref[...]
    ah_ref[...] = ah
    z = jnp.zeros_like(eh[:, :H])
    db_tab_ref[0] = jnp.concatenate([dh[:, :H], bh[:, :H]], axis=1)
    db_tab_ref[1] = jnp.concatenate([dh[:, H:], bh[:, H:]], axis=1)
    eh_tab_ref[0] = jnp.concatenate([eh[:, :H], z], axis=1)
    eh_tab_ref[1] = jnp.concatenate([eh[:, H:], z], axis=1)


def _node_transforms(h, Aw, Ab, Bw, Bb, Dw, Db, Ew, Eb):
    R = 1000
    grid = (N // R,)
    wspec = pl.BlockSpec((D, D), lambda i: (0, 0))
    bspec = pl.BlockSpec((D,), lambda i: (0,))
    return pl.pallas_call(
        _a1_body,
        grid=grid,
        in_specs=[pl.BlockSpec((R, D), lambda i: (i, 0)),
                  wspec, bspec, wspec, bspec, wspec, bspec, wspec, bspec],
        out_specs=[pl.BlockSpec((R, D), lambda i: (i, 0)),
                   pl.BlockSpec((2, R, D), lambda i: (0, i, 0)),
                   pl.BlockSpec((2, R, D), lambda i: (0, i, 0))],
        out_shape=[jax.ShapeDtypeStruct((N, D), jnp.float32),
                   jax.ShapeDtypeStruct((2, N, D), jnp.float32),
                   jax.ShapeDtypeStruct((2, N, D), jnp.float32)],
    )(h, Aw, Ab, Bw, Bb, Dw, Db, Ew, Eb)


def _a2_body(e2_ref, wp_ref, cbp_ref, ce_ref):
    e2 = e2_ref[...].astype(jnp.bfloat16)
    wp = wp_ref[...].astype(jnp.bfloat16)
    ce_ref[0] = lax.dot_general(e2, wp[0], (((1,), (0,)), ((), ())),
                                preferred_element_type=jnp.float32) + cbp_ref[0]
    ce_ref[1] = lax.dot_general(e2, wp[1], (((1,), (0,)), ((), ())),
                                preferred_element_type=jnp.float32) + cbp_ref[1]


def _edge_transform(e2, Wp, Cbp):
    T2 = 2000
    return pl.pallas_call(
        _a2_body,
        grid=(E // 2 // T2,),
        in_specs=[pl.BlockSpec((T2, 2 * D), lambda i: (i, 0)),
                  pl.BlockSpec((2, 2 * D, D), lambda i: (0, 0, 0)),
                  pl.BlockSpec((2, D), lambda i: (0, 0))],
        out_specs=pl.BlockSpec((2, T2, D), lambda i: (0, i, 0)),
        out_shape=jax.ShapeDtypeStruct((2, E // 2, D), jnp.float32),
    )(e2, Wp, Cbp)


def _sc_body(db_hbm, eh_hbm, ce_hbm, src_hbm, dst_hbm,
             enew_hbm, acc_hbm,
             acc,
             src_v0, dst_v0, dstg_v0, db_b0, eh_b0, ce_b0, out_b0, ms_b0,
             srcB0, dstB0, sem_i0,
             src_v1, dst_v1, dstg_v1, db_b1, eh_b1, ce_b1, out_b1, ms_b1,
             srcB1, dstB1, sem_i1,
             sem_g0, sem_g1, sem_w0, sem_w1):
    c = lax.axis_index("c")
    s = lax.axis_index("s")
    cN = c * N
    cP = c * NPAD
    cE2 = c * (E // 2)

    bufs = ((src_v0, dst_v0, dstg_v0, db_b0, eh_b0, ce_b0, out_b0, ms_b0,
             sem_g0, sem_w0),
            (src_v1, dst_v1, dstg_v1, db_b1, eh_b1, ce_b1, out_b1, ms_b1,
             sem_g1, sem_w1))
    ibufs = ((srcB0, dstB0, sem_i0), (srcB1, dstB1, sem_i1))

    def zb_row(r, _):
        for j in range(D // 16):
            ms_b0[r, pl.ds(j * 16, 16)] = jnp.zeros((16,), jnp.float32)
        return 0
    lax.fori_loop(0, CHUNK, zb_row, 0)
    for k in range(RPT // CHUNK):
        pltpu.sync_copy(ms_b0, acc.at[pl.ds(s * RPT + k * CHUNK, CHUNK)])
    plsc.subcore_barrier()

    base0 = s * EPT
    base20 = s * (EPT // 2)

    def issue_idx(b, k):
        sb, db_, si = ibufs[b]
        base = base0 + k * CHUNK
        pltpu.async_copy(src_hbm.at[pl.ds(base, CHUNK)], sb, si)
        pltpu.async_copy(dst_hbm.at[pl.ds(base, CHUNK)], db_, si)

    def wait_idx(b):
        sb, db_, si = ibufs[b]
        pltpu.make_async_copy(src_hbm.at[pl.ds(0, CHUNK)], sb, si).wait()
        pltpu.make_async_copy(dst_hbm.at[pl.ds(0, CHUNK)], db_, si).wait()

    def issue_loads(b, k):
        sv, dv, gv, bdb, beh, bce, _, _, sg, _ = bufs[b]
        sb, db_, si = ibufs[b]
        base2 = base20 + k * (CHUNK // 2)
        wait_idx(b)
        for j in range(CHUNK // 16):
            sl = pl.ds(j * 16, 16)
            sv[sl] = sb[sl] + cN
            rd = db_[sl]
            dv[sl] = rd
            gv[sl] = rd + cN
        pltpu.async_copy(db_hbm.at[sv], bdb, sg)
        pltpu.async_copy(eh_hbm.at[gv], beh, sg)
        pltpu.async_copy(ce_hbm.at[pl.ds(cE2 + base2, CHUNK // 2)], bce, sg)

    def wait_loads(b, k):
        sv, dv, gv, bdb, beh, bce, _, _, sg, _ = bufs[b]
        base2 = base20 + k * (CHUNK // 2)
        pltpu.make_async_copy(db_hbm.at[sv], bdb, sg).wait()
        pltpu.make_async_copy(eh_hbm.at[gv], beh, sg).wait()
        pltpu.make_async_copy(
            ce_hbm.at[pl.ds(cE2 + base2, CHUNK // 2)], bce, sg).wait()

    def finish_chunk(b, k):
        _, dv, _, _, _, _, bout, bms, _, sw = bufs[b]
        base2 = base20 + k * (CHUNK // 2)
        pltpu.sync_copy(bms, acc.at[dv], add=True)
        pltpu.async_copy(bout, enew_hbm.at[pl.ds(cE2 + base2, CHUNK // 2)],
                         sw)

    def drain_enew(b):
        _, _, _, _, _, _, bout, _, _, sw = bufs[b]
        pltpu.make_async_copy(bout, enew_hbm.at[pl.ds(0, CHUNK // 2)],
                              sw).wait()

    def compute(b):
        _, _, _, bdb, beh, bce, bout, bms, _, _ = bufs[b]

        @plsc.parallel_loop(0, CHUNK // 2, 1, unroll=4)
        def pair_body(rr):
            for p in range(2):
                r = 2 * rr + p
                for j in range(H // 16):
                    sl_de = pl.ds(j * 16, 16)
                    sl_b = pl.ds(H + j * 16, 16)
                    sl_ce = pl.ds(p * H + j * 16, 16)
                    x = bdb[r, sl_de] + beh[r, sl_de] + bce[rr, sl_ce]
                    bout[rr, sl_ce] = x
                    sg = 1.0 / (1.0 + jnp.exp(-x))
                    bms[r, sl_b] = sg
                    bms[r, sl_de] = bdb[r, sl_b] * sg

    NCHUNK = EPT // CHUNK
    NG2 = NCHUNK // 2

    def body2(g, _):
        k0 = 2 * g
        wait_loads(0, k0)

        @pl.when(g > 0)
        def _():
            drain_enew(0)
        compute(0)
        finish_chunk(0, k0)

        @pl.when(k0 + 2 < NCHUNK)
        def _():
            issue_loads(0, k0 + 2)

        @pl.when(k0 + 4 < NCHUNK)
        def _():
            issue_idx(0, k0 + 4)
        wait_loads(1, k0 + 1)

        @pl.when(g > 0)
        def _():
            drain_enew(1)
        compute(1)
        finish_chunk(1, k0 + 1)

        @pl.when(k0 + 3 < NCHUNK)
        def _():
            issue_loads(1, k0 + 3)

        @pl.when(k0 + 5 < NCHUNK)
        def _():
            issue_idx(1, k0 + 5)
        return 0

    issue_idx(0, 0)
    issue_idx(1, 1)
    issue_loads(0, 0)
    issue_idx(0, 2)
    issue_loads(1, 1)
    issue_idx(1, 3)
    lax.fori_loop(0, NG2, body2, 0)
    drain_enew(0)
    if NCHUNK % 2:
        tk = 2 * NG2
        wait_loads(0, tk)
        compute(0)
        finish_chunk(0, tk)
        drain_enew(0)
    drain_enew(1)

    plsc.subcore_barrier()
    pltpu.sync_copy(acc.at[pl.ds(s * RPT, RPT)],
                    acc_hbm.at[pl.ds(cP + s * RPT, RPT)])


def _sc_stage(db, eh, ce, src, dst):
    mesh = plsc.VectorSubcoreMesh(core_axis_name="c", subcore_axis_name="s",
                                  num_cores=NC, num_subcores=NS)
    f = pl.kernel(
        _sc_body,
        out_type=[jax.ShapeDtypeStruct((E, D), jnp.float32),
                  jax.ShapeDtypeStruct((2 * NPAD, D), jnp.float32)],
        mesh=mesh,
        scratch_types=(
            [pltpu.VMEM_SHARED((NPAD, D), jnp.float32)]
            + 2 * [pltpu.VMEM((CHUNK,), jnp.int32),
                   pltpu.VMEM((CHUNK,), jnp.int32),
                   pltpu.VMEM((CHUNK,), jnp.int32),
                   pltpu.VMEM((CHUNK, D), jnp.float32),
                   pltpu.VMEM((CHUNK, D), jnp.float32),
                   pltpu.VMEM((CHUNK // 2, D), jnp.float32),
                   pltpu.VMEM((CHUNK // 2, D), jnp.float32),
                   pltpu.VMEM((CHUNK, D), jnp.float32),
                   pltpu.VMEM((CHUNK,), jnp.int32),
                   pltpu.VMEM((CHUNK,), jnp.int32),
                   pltpu.SemaphoreType.DMA]
            + [pltpu.SemaphoreType.DMA,
               pltpu.SemaphoreType.DMA,
               pltpu.SemaphoreType.DMA,
               pltpu.SemaphoreType.DMA]
        ),
    )
    return f(db, eh, ce, src, dst)


def _c1_body(h_ref, ah_ref, acc_ref, sc_ref, bi_ref, out_ref):
    sh = jnp.concatenate([acc_ref[0, :N, :H], acc_ref[1, :N, :H]], axis=1)
    ss = jnp.concatenate([acc_ref[0, :N, H:], acc_ref[1, :N, H:]], axis=1)
    hn = ah_ref[...] + sh / (ss + 1e-6)
    mu = jnp.mean(hn, axis=0)
    var = jnp.mean(hn * hn, axis=0) - mu * mu
    bn = (hn - mu) / jnp.sqrt(var + 1e-5) * sc_ref[...] + bi_ref[...]
    out_ref[...] = h_ref[...] + jnp.maximum(bn, 0.0)


def _finalize_h(h, ah, acc, scale, bias):
    return pl.pallas_call(
        _c1_body,
        out_shape=jax.ShapeDtypeStruct((N, D), jnp.float32),
    )(h, ah, acc, scale, bias)


def _c2a_body(en_ref, st_ref):
    i = pl.program_id(0)

    @pl.when(i == 0)
    def _():
        st_ref[...] = jnp.zeros_like(st_ref)
    en0 = en_ref[0]
    en1 = en_ref[1]
    st_ref[0, :D] += jnp.sum(en0, axis=0)
    st_ref[0, D:] += jnp.sum(en1, axis=0)
    st_ref[1, :D] += jnp.sum(en0 * en0, axis=0)
    st_ref[1, D:] += jnp.sum(en1 * en1, axis=0)


def _e_stats(enew):
    T2 = 4000
    return pl.pallas_call(
        _c2a_body,
        grid=(E // 2 // T2,),
        in_specs=[pl.BlockSpec((2, T2, D), lambda i: (0, i, 0))],
        out_specs=pl.BlockSpec((8, 2 * D), lambda i: (0, 0)),
        out_shape=jax.ShapeDtypeStruct((8, 2 * D), jnp.float32),
    )(enew)


def _c2_body(e2_ref, en_ref, st_ref, sc_ref, bi_ref, out_ref):
    en0 = en_ref[0]
    en1 = en_ref[1]
    en = jnp.concatenate([en0[:, :H], en1[:, :H], en0[:, H:], en1[:, H:]],
                         axis=1)
    s0 = st_ref[0, :D]
    s1 = st_ref[0, D:]
    q0 = st_ref[1, :D]
    q1 = st_ref[1, D:]
    mu = jnp.concatenate([s0[:H] + s0[H:], s1[:H] + s1[H:]]) * (1.0 / E)
    msq = jnp.concatenate([q0[:H] + q0[H:], q1[:H] + q1[H:]]) * (1.0 / E)
    var = msq - mu * mu
    rstd = 1.0 / jnp.sqrt(var + 1e-5)
    mu2 = jnp.concatenate([mu, mu])
    rstd2 = jnp.concatenate([rstd, rstd])
    sc2 = jnp.concatenate([sc_ref[...], sc_ref[...]])
    bi2 = jnp.concatenate([bi_ref[...], bi_ref[...]])
    bn = (en - mu2) * rstd2 * sc2 + bi2
    out_ref[...] = e2_ref[...] + jnp.maximum(bn, 0.0)


def _finalize_e(e2, enew, stats, scale, bias):
    T2 = 2000
    return pl.pallas_call(
        _c2_body,
        grid=(E // 2 // T2,),
        in_specs=[pl.BlockSpec((T2, 2 * D), lambda i: (i, 0)),
                  pl.BlockSpec((2, T2, D), lambda i: (0, i, 0)),
                  pl.BlockSpec((8, 2 * D), lambda i: (0, 0)),
                  pl.BlockSpec((D,), lambda i: (0,)),
                  pl.BlockSpec((D,), lambda i: (0,))],
        out_specs=pl.BlockSpec((T2, 2 * D), lambda i: (i, 0)),
        out_shape=jax.ShapeDtypeStruct((E // 2, 2 * D), jnp.float32),
    )(e2, enew, stats, scale, bias)


@jax.jit
def _run(h, e, edge_index, Aw, Ab, Bw, Bb, Cw, Cb, Dw, Db, Ew, Eb,
         bnh_scale, bnh_bias, bne_scale, bne_bias):
    src = edge_index[0].astype(jnp.int32)
    dst = edge_index[1].astype(jnp.int32)

    wt = Cw.T
    z = jnp.zeros((D, H), jnp.float32)
    wp_list = []
    cbp_list = []
    for c in range(2):
        blk = wt[:, c * H:(c + 1) * H]
        top = jnp.concatenate([blk, z], axis=1)
        bot = jnp.concatenate([z, blk], axis=1)
        wp_list.append(jnp.concatenate([top, bot], axis=0))
        cbp_list.append(jnp.tile(Cb[c * H:(c + 1) * H], 2))
    Wp = jnp.stack(wp_list)
    Cbp = jnp.stack(cbp_list)
    e2 = e.reshape(E // 2, 2 * D)

    ah, db_tab, eh_tab = _node_transforms(h, Aw, Ab, Bw, Bb, Dw, Db, Ew, Eb)
    ce = _edge_transform(e2, Wp, Cbp)

    enew, accs = _sc_stage(db_tab.reshape(2 * N, D),
                           eh_tab.reshape(2 * N, D),
                           ce.reshape(E, D), src, dst)

    enew3 = enew.reshape(2, E // 2, D)
    stats = _e_stats(enew3)
    h_out = _finalize_h(h, ah, accs.reshape(2, NPAD, D), bnh_scale, bnh_bias)
    e_out2 = _finalize_e(e2, enew3, stats, bne_scale, bne_bias)
    return h_out, e_out2.reshape(E, D)


def kernel(h, e, edge_index, Aw, Ab, Bw, Bb, Cw, Cb, Dw, Db, Ew, Eb,
           bnh_scale, bnh_bias, bne_scale, bne_bias):
    return _run(h, e, edge_index, Aw, Ab, Bw, Bb, Cw, Cb, Dw, Db, Ew, Eb,
                bnh_scale, bnh_bias, bne_scale, bne_bias)
